# baseline (speedup 1.0000x reference)
import jax
import jax.numpy as jnp
from jax.experimental import pallas as pl

MAX_LEN = 4096
BATCH = 2
D_MODEL = 1024

BLOCK_P = 512


def _add_body(x_ref, e_ref, o_ref):
    e = e_ref[...]
    for b in range(BATCH):
        sl = slice(b * D_MODEL, (b + 1) * D_MODEL)
        o_ref[:, sl] = x_ref[:, sl] + e


def kernel(x, emb_table):
    x2 = x.reshape(MAX_LEN, BATCH * D_MODEL)
    out = pl.pallas_call(
        _add_body,
        grid=(MAX_LEN // BLOCK_P,),
        in_specs=[
            pl.BlockSpec((BLOCK_P, BATCH * D_MODEL), lambda i: (i, 0)),
            pl.BlockSpec((BLOCK_P, D_MODEL), lambda i: (i, 0)),
        ],
        out_specs=pl.BlockSpec((BLOCK_P, BATCH * D_MODEL), lambda i: (i, 0)),
        out_shape=jax.ShapeDtypeStruct((MAX_LEN, BATCH * D_MODEL), jnp.float32),
    )(x2, emb_table)
    return out.reshape(MAX_LEN, BATCH, D_MODEL)


# speedup vs baseline: 3.3671x; 3.3671x over previous
import jax
import jax.numpy as jnp
from jax.experimental import pallas as pl

MAX_LEN = 4096
BATCH = 2
D_MODEL = 1024

BLOCK_P = 256


def _add_body(x_ref, e_ref, o_ref):
    o_ref[...] = x_ref[...] + e_ref[...][:, None, :]


def kernel(x, emb_table):
    grid = (MAX_LEN // BLOCK_P,)
    return pl.pallas_call(
        _add_body,
        grid=grid,
        in_specs=[
            pl.BlockSpec((BLOCK_P, BATCH, D_MODEL), lambda i: (i, 0, 0)),
            pl.BlockSpec((BLOCK_P, D_MODEL), lambda i: (i, 0)),
        ],
        out_specs=pl.BlockSpec((BLOCK_P, BATCH, D_MODEL), lambda i: (i, 0, 0)),
        out_shape=jax.ShapeDtypeStruct((MAX_LEN, BATCH, D_MODEL), jnp.float32),
    )(x, emb_table)


# speedup vs baseline: 3.7190x; 1.1045x over previous
import jax
import jax.numpy as jnp
from jax.experimental import pallas as pl

MAX_LEN = 4096
BATCH = 2
D_MODEL = 1024

BLOCK_P = 1024


def _add_body(x_ref, e_ref, o_ref):
    o_ref[...] = x_ref[...] + e_ref[...][:, None, :]


def kernel(x, emb_table):
    grid = (MAX_LEN // BLOCK_P,)
    return pl.pallas_call(
        _add_body,
        grid=grid,
        in_specs=[
            pl.BlockSpec((BLOCK_P, BATCH, D_MODEL), lambda i: (i, 0, 0)),
            pl.BlockSpec((BLOCK_P, D_MODEL), lambda i: (i, 0)),
        ],
        out_specs=pl.BlockSpec((BLOCK_P, BATCH, D_MODEL), lambda i: (i, 0, 0)),
        out_shape=jax.ShapeDtypeStruct((MAX_LEN, BATCH, D_MODEL), jnp.float32),
    )(x, emb_table)


# speedup vs baseline: 4.1029x; 1.1032x over previous
import jax
import jax.numpy as jnp
from jax.experimental import pallas as pl

MAX_LEN = 4096
BATCH = 2
D_MODEL = 1024

BLOCK_P = 1024


def _add_body(x_ref, e_ref, o_ref):
    e = e_ref[...]
    for b in range(BATCH):
        o_ref[:, b, :] = x_ref[:, b, :] + e


def kernel(x, emb_table):
    grid = (MAX_LEN // BLOCK_P,)
    return pl.pallas_call(
        _add_body,
        grid=grid,
        in_specs=[
            pl.BlockSpec((BLOCK_P, BATCH, D_MODEL), lambda i: (i, 0, 0)),
            pl.BlockSpec((BLOCK_P, D_MODEL), lambda i: (i, 0)),
        ],
        out_specs=pl.BlockSpec((BLOCK_P, BATCH, D_MODEL), lambda i: (i, 0, 0)),
        out_shape=jax.ShapeDtypeStruct((MAX_LEN, BATCH, D_MODEL), jnp.float32),
    )(x, emb_table)


# speedup vs baseline: 4.1661x; 1.0154x over previous
import jax
import jax.numpy as jnp
from jax.experimental import pallas as pl
from jax.experimental.pallas import tpu as pltpu

MAX_LEN = 4096
BATCH = 2
D_MODEL = 1024

BLOCK_P = 1024


def _add_body(x_ref, e_ref, o_ref):
    e = e_ref[...]
    for b in range(BATCH):
        o_ref[:, b, :] = x_ref[:, b, :] + e


def kernel(x, emb_table):
    grid = (MAX_LEN // BLOCK_P,)
    return pl.pallas_call(
        _add_body,
        grid=grid,
        in_specs=[
            pl.BlockSpec((BLOCK_P, BATCH, D_MODEL), lambda i: (i, 0, 0)),
            pl.BlockSpec((BLOCK_P, D_MODEL), lambda i: (i, 0)),
        ],
        out_specs=pl.BlockSpec((BLOCK_P, BATCH, D_MODEL), lambda i: (i, 0, 0)),
        out_shape=jax.ShapeDtypeStruct((MAX_LEN, BATCH, D_MODEL), jnp.float32),
        compiler_params=pltpu.CompilerParams(
            dimension_semantics=("parallel",),
        ),
    )(x, emb_table)


# speedup vs baseline: 4.1669x; 1.0002x over previous
import jax
import jax.numpy as jnp
from jax.experimental import pallas as pl
from jax.experimental.pallas import tpu as pltpu

MAX_LEN = 4096
BATCH = 2
D_MODEL = 1024

BLOCK_P = 512


def _add_body(x_ref, e_ref, o_ref):
    e = e_ref[...]
    for b in range(BATCH):
        o_ref[:, b, :] = x_ref[:, b, :] + e


def kernel(x, emb_table):
    grid = (MAX_LEN // BLOCK_P,)
    return pl.pallas_call(
        _add_body,
        grid=grid,
        in_specs=[
            pl.BlockSpec((BLOCK_P, BATCH, D_MODEL), lambda i: (i, 0, 0)),
            pl.BlockSpec((BLOCK_P, D_MODEL), lambda i: (i, 0)),
        ],
        out_specs=pl.BlockSpec((BLOCK_P, BATCH, D_MODEL), lambda i: (i, 0, 0)),
        out_shape=jax.ShapeDtypeStruct((MAX_LEN, BATCH, D_MODEL), jnp.float32),
        compiler_params=pltpu.CompilerParams(
            dimension_semantics=("parallel",),
        ),
    )(x, emb_table)
